# Initial kernel scaffold; baseline (speedup 1.0000x reference)
import jax
import jax.numpy as jnp
from jax.experimental import pallas as pl


def kernel(x, edge_index, W0, b0, Wl, attl, attr, Wp, Wn, Wc, Wout, bout):
    raise NotImplementedError("write your pallas kernel here")


# speedup vs baseline: 1.0629x; 1.0629x over previous
import math

import jax
import jax.numpy as jnp
from jax.experimental import pallas as pl

N_NODES = 10000
N_EDGES = 320000
C = 128
NUM_CLASSES = 40
NUM_LAYERS = 8
ALPHA = 0.1
THETA = 0.5


def _head_body(h_ref, w_ref, b_ref, o_ref):
    logits = jnp.dot(h_ref[...], w_ref[...], preferred_element_type=jnp.float32)
    logits = logits + b_ref[...]
    m = jnp.max(logits, axis=-1, keepdims=True)
    z = logits - m
    lse = jnp.log(jnp.sum(jnp.exp(z), axis=-1, keepdims=True))
    o_ref[...] = z - lse


def _head(h, Wout, bout):
    n = h.shape[0]
    wp = jnp.zeros((C, 128), jnp.float32).at[:, :NUM_CLASSES].set(Wout)
    bp = jnp.zeros((1, 128), jnp.float32).at[0, :NUM_CLASSES].set(bout)
    bp = bp.at[0, NUM_CLASSES:].set(-1e30)
    out = pl.pallas_call(
        _head_body,
        out_shape=jax.ShapeDtypeStruct((n, 128), jnp.float32),
        grid=(5,),
        in_specs=[
            pl.BlockSpec((n // 5, C), lambda i: (i, 0)),
            pl.BlockSpec((C, 128), lambda i: (0, 0)),
            pl.BlockSpec((1, 128), lambda i: (0, 0)),
        ],
        out_specs=pl.BlockSpec((n // 5, 128), lambda i: (i, 0)),
    )(h, wp, bp)
    return out[:, :NUM_CLASSES]


def kernel(x, edge_index, W0, b0, Wl, attl, attr, Wp, Wn, Wc, Wout, bout):
    n = x.shape[0]
    loop = jnp.arange(n, dtype=edge_index.dtype)
    src = jnp.concatenate([edge_index[0], loop])
    dst = jnp.concatenate([edge_index[1], loop])
    ew = jnp.ones(src.shape[0], jnp.float32)
    deg = jax.ops.segment_sum(ew, dst, num_segments=n)
    dis = jnp.where(deg > 0, 1.0 / jnp.sqrt(jnp.where(deg > 0, deg, 1.0)), 0.0)
    ew = dis[src] * ew * dis[dst]

    h = jax.nn.relu(x @ W0 + b0)
    x0 = h
    sigmas = []
    for l in range(NUM_LAYERS):
        beta = math.log(THETA / (l + 1) + 1.0)
        al = Wl[l] @ attl[l]
        ar = Wl[l] @ attr[l]
        sl = h @ al
        sr = h @ ar
        xp = h @ Wp[l]
        xn = h @ Wn[l]
        s = jax.nn.sigmoid(jax.nn.leaky_relu(sl[src] + sr[dst], 0.2))
        w1 = ew * s
        w0 = ew - w1
        c = jax.ops.segment_sum(w0, dst, num_segments=n)
        agg1 = jax.ops.segment_sum(w1[:, None] * xp[src], dst, num_segments=n)
        agg = agg1 + xn * c[:, None]
        xo = agg * (1.0 - ALPHA) + ALPHA * x0
        Wcp = (1.0 - beta) * jnp.eye(C, dtype=jnp.float32) + beta * Wc[l]
        h = jax.nn.relu(xo @ Wcp)
        sigmas.append(s)
    return _head(h, Wout, bout), tuple(sigmas)


# speedup vs baseline: 10.0773x; 9.4812x over previous
import functools
import math

import jax
import jax.numpy as jnp
from jax import lax
from jax.experimental import pallas as pl
from jax.experimental.pallas import tpu as pltpu
from jax.experimental.pallas import tpu_sc as plsc

N_NODES = 10000
N_EDGES = 320000
C = 128
NUM_CLASSES = 40
NUM_LAYERS = 8
ALPHA = 0.1
THETA = 0.5

NW = 32
EB = 128
NCH = 80
EPW = EB * NCH
EPAD = NW * EPW
NZB = 1000


def _sc_mesh():
    return plsc.VectorSubcoreMesh(core_axis_name="c", subcore_axis_name="s")


def _deg_body(dst_hbm, ones_hbm, zc_hbm, out_hbm, dstv, onesv, acc_sh, sem):
    core = lax.axis_index("c")
    sub = lax.axis_index("s")
    wid = sub * 2 + core

    @pl.when(sub == 0)
    def _zero():
        pltpu.sync_copy(zc_hbm, acc_sh)

    plsc.subcore_barrier()

    def chunk(k, carry):
        base = wid * EPW + k * EB
        pltpu.sync_copy(dst_hbm.at[pl.ds(base, EB)], dstv)
        pltpu.sync_copy(ones_hbm.at[pl.ds(base, EB)], onesv)
        pltpu.sync_copy(onesv, acc_sh.at[dstv], add=True)
        return carry

    lax.fori_loop(0, NCH, chunk, 0)
    plsc.subcore_barrier()

    @pl.when(sub == 0)
    def _out():
        pltpu.sync_copy(acc_sh, out_hbm.at[core, 0])


def _deg_kernel(dstp, onesp, zc):
    f = functools.partial(
        pl.kernel,
        mesh=_sc_mesh(),
        out_type=jax.ShapeDtypeStruct((2, 1, N_NODES), jnp.float32),
        scratch_types=[
            pltpu.VMEM((EB,), jnp.int32),
            pltpu.VMEM((EB,), jnp.float32),
            pltpu.VMEM_SHARED((N_NODES,), jnp.float32),
            pltpu.SemaphoreType.DMA,
        ],
    )(_deg_body)
    return f(dstp, onesp, zc)


def _ew_body(src_hbm, dst_hbm, ones_hbm, dis_hbm, out_hbm,
             srcv, dstv, onesv, dav, dbv, ewv, sem1, sem2):
    core = lax.axis_index("c")
    sub = lax.axis_index("s")
    wid = sub * 2 + core

    def chunk(k, carry):
        base = wid * EPW + k * EB
        pltpu.sync_copy(src_hbm.at[pl.ds(base, EB)], srcv)
        pltpu.sync_copy(dst_hbm.at[pl.ds(base, EB)], dstv)
        pltpu.sync_copy(ones_hbm.at[pl.ds(base, EB)], onesv)
        cp1 = pltpu.async_copy(dis_hbm.at[srcv], dav, sem1)
        cp2 = pltpu.async_copy(dis_hbm.at[dstv], dbv, sem2)
        cp1.wait()
        cp2.wait()
        for j in range(EB // 16):
            sl = pl.ds(j * 16, 16)
            ewv[sl] = dav[sl] * dbv[sl] * onesv[sl]
        pltpu.sync_copy(ewv, out_hbm.at[pl.ds(base, EB)])
        return carry

    lax.fori_loop(0, NCH, chunk, 0)


def _ew_kernel(srcp, dstp, onesp, dis):
    f = functools.partial(
        pl.kernel,
        mesh=_sc_mesh(),
        out_type=jax.ShapeDtypeStruct((EPAD,), jnp.float32),
        scratch_types=[
            pltpu.VMEM((EB,), jnp.int32),
            pltpu.VMEM((EB,), jnp.int32),
            pltpu.VMEM((EB,), jnp.float32),
            pltpu.VMEM((EB,), jnp.float32),
            pltpu.VMEM((EB,), jnp.float32),
            pltpu.VMEM((EB,), jnp.float32),
            pltpu.SemaphoreType.DMA,
            pltpu.SemaphoreType.DMA,
        ],
    )(_ew_body)
    return f(srcp, dstp, onesp, dis)


def _edge_body(src_hbm, dst_hbm, ew_hbm, sl_hbm, sr_hbm, xp_hbm,
               zc_hbm, zagg_hbm,
               s_hbm, c_hbm, agg_hbm,
               srcv, dstv, eww, slv, srv, sv, w1v, w0v, rows,
               c_sh, agg_sh, sem1, sem2, sem3):
    core = lax.axis_index("c")
    sub = lax.axis_index("s")
    wid = sub * 2 + core

    @pl.when(sub == 0)
    def _zero_c():
        pltpu.sync_copy(zc_hbm, c_sh)

    @pl.when(sub < 10)
    def _zero_agg():
        pltpu.sync_copy(zagg_hbm.at[pl.ds(sub * NZB, NZB)],
                        agg_sh.at[pl.ds(sub * NZB, NZB)])

    plsc.subcore_barrier()

    def chunk(k, carry):
        base = wid * EPW + k * EB
        pltpu.sync_copy(src_hbm.at[pl.ds(base, EB)], srcv)
        pltpu.sync_copy(dst_hbm.at[pl.ds(base, EB)], dstv)
        pltpu.sync_copy(ew_hbm.at[pl.ds(base, EB)], eww)
        cp1 = pltpu.async_copy(sl_hbm.at[srcv], slv, sem1)
        cp2 = pltpu.async_copy(sr_hbm.at[dstv], srv, sem2)
        cp3 = pltpu.async_copy(xp_hbm.at[srcv], rows, sem3)
        cp1.wait()
        cp2.wait()
        for j in range(EB // 16):
            sl = pl.ds(j * 16, 16)
            t = slv[sl] + srv[sl]
            t = jnp.where(t >= 0.0, t, 0.2 * t)
            s = 1.0 / (1.0 + jnp.exp(-t))
            w1 = eww[sl] * s
            sv[sl] = s
            w1v[sl] = w1
            w0v[sl] = eww[sl] - w1
        pltpu.sync_copy(sv, s_hbm.at[pl.ds(base, EB)])
        pltpu.sync_copy(w0v, c_sh.at[dstv], add=True)
        cp3.wait()

        def scale(g, carry2):
            w16 = w1v[pl.ds(g * 16, 16)]
            for r in range(16):
                w = jnp.full((16,), w16[r], jnp.float32)
                i = g * 16 + r
                for j in range(C // 16):
                    sl2 = pl.ds(j * 16, 16)
                    rows[i, sl2] = rows[i, sl2] * w
            return carry2

        lax.fori_loop(0, EB // 16, scale, 0)
        pltpu.sync_copy(rows, agg_sh.at[dstv], add=True)
        return carry

    lax.fori_loop(0, NCH, chunk, 0)
    plsc.subcore_barrier()

    @pl.when(sub < 10)
    def _out_agg():
        pltpu.sync_copy(agg_sh.at[pl.ds(sub * NZB, NZB)],
                        agg_hbm.at[core, pl.ds(sub * NZB, NZB)])

    @pl.when(sub == 0)
    def _out_c():
        pltpu.sync_copy(c_sh, c_hbm.at[core, 0])


def _edge_kernel(srcp, dstp, ewp, sl, sr, xp, zc, zagg):
    f = functools.partial(
        pl.kernel,
        mesh=_sc_mesh(),
        out_type=(
            jax.ShapeDtypeStruct((EPAD,), jnp.float32),
            jax.ShapeDtypeStruct((2, 1, N_NODES), jnp.float32),
            jax.ShapeDtypeStruct((2, N_NODES, C), jnp.float32),
        ),
        scratch_types=[
            pltpu.VMEM((EB,), jnp.int32),
            pltpu.VMEM((EB,), jnp.int32),
            pltpu.VMEM((EB,), jnp.float32),
            pltpu.VMEM((EB,), jnp.float32),
            pltpu.VMEM((EB,), jnp.float32),
            pltpu.VMEM((EB,), jnp.float32),
            pltpu.VMEM((EB,), jnp.float32),
            pltpu.VMEM((EB,), jnp.float32),
            pltpu.VMEM((EB, C), jnp.float32),
            pltpu.VMEM_SHARED((N_NODES,), jnp.float32),
            pltpu.VMEM_SHARED((N_NODES, C), jnp.float32),
            pltpu.SemaphoreType.DMA,
            pltpu.SemaphoreType.DMA,
            pltpu.SemaphoreType.DMA,
        ],
    )(_edge_body)
    return f(srcp, dstp, ewp, sl, sr, xp, zc, zagg)


def _prolog_body(x_ref, w0_ref, b0_ref, degp_ref, h_ref, dis_ref):
    h = jnp.dot(x_ref[...], w0_ref[...], preferred_element_type=jnp.float32)
    h_ref[...] = jax.nn.relu(h + b0_ref[...])
    deg = degp_ref[0] + degp_ref[1] + 1.0
    dis_ref[...] = lax.rsqrt(deg)


def _prolog(x, W0, b0, degpart):
    nb = N_NODES // 5
    return pl.pallas_call(
        _prolog_body,
        out_shape=(
            jax.ShapeDtypeStruct((N_NODES, C), jnp.float32),
            jax.ShapeDtypeStruct((N_NODES, 1), jnp.float32),
        ),
        grid=(5,),
        in_specs=[
            pl.BlockSpec((nb, C), lambda i: (i, 0)),
            pl.BlockSpec((C, C), lambda i: (0, 0)),
            pl.BlockSpec((1, C), lambda i: (0, 0)),
            pl.BlockSpec((2, nb, 1), lambda i: (0, i, 0)),
        ],
        out_specs=(
            pl.BlockSpec((nb, C), lambda i: (i, 0)),
            pl.BlockSpec((nb, 1), lambda i: (i, 0)),
        ),
    )(x, W0, b0.reshape(1, C), degpart)


def _proj_body(h_ref, wl_ref, attlr_ref, wp_ref, wn_ref,
               xp_ref, xn_ref, svec_ref):
    h = h_ref[...]
    alr = lax.dot_general(wl_ref[...], attlr_ref[...],
                          (((1,), (1,)), ((), ())),
                          preferred_element_type=jnp.float32)
    svec_ref[...] = jnp.dot(h, alr, preferred_element_type=jnp.float32)
    xp_ref[...] = jnp.dot(h, wp_ref[...], preferred_element_type=jnp.float32)
    xn_ref[...] = jnp.dot(h, wn_ref[...], preferred_element_type=jnp.float32)


def _proj(h, wl, attlr, wp, wn):
    nb = N_NODES // 5
    return pl.pallas_call(
        _proj_body,
        out_shape=(
            jax.ShapeDtypeStruct((N_NODES, C), jnp.float32),
            jax.ShapeDtypeStruct((N_NODES, C), jnp.float32),
            jax.ShapeDtypeStruct((N_NODES, 2), jnp.float32),
        ),
        grid=(5,),
        in_specs=[
            pl.BlockSpec((nb, C), lambda i: (i, 0)),
            pl.BlockSpec((C, C), lambda i: (0, 0)),
            pl.BlockSpec((2, C), lambda i: (0, 0)),
            pl.BlockSpec((C, C), lambda i: (0, 0)),
            pl.BlockSpec((C, C), lambda i: (0, 0)),
        ],
        out_specs=(
            pl.BlockSpec((nb, C), lambda i: (i, 0)),
            pl.BlockSpec((nb, C), lambda i: (i, 0)),
            pl.BlockSpec((nb, 2), lambda i: (i, 0)),
        ),
    )(h, wl, attlr, wp, wn)


def _update_body(beta, aggp_ref, cp_ref, xp_ref, xn_ref, svec_ref, dis_ref,
                 x0_ref, wc_ref, h_ref, sself_ref):
    t = svec_ref[:, 0:1] + svec_ref[:, 1:2]
    t = jnp.where(t >= 0.0, t, 0.2 * t)
    s = 1.0 / (1.0 + jnp.exp(-t))
    sself_ref[...] = s
    dis2 = dis_ref[...] * dis_ref[...]
    csum = cp_ref[0] + cp_ref[1]
    xn = xn_ref[...]
    agg = (aggp_ref[0] + aggp_ref[1]
           + xn * (csum + dis2 * (1.0 - s))
           + xp_ref[...] * (dis2 * s))
    xo = agg * (1.0 - ALPHA) + ALPHA * x0_ref[...]
    xw = jnp.dot(xo, wc_ref[...], preferred_element_type=jnp.float32)
    h_ref[...] = jax.nn.relu((1.0 - beta) * xo + beta * xw)


def _update(beta, aggpart, cpart, xp, xn, svec, dis, x0, wc):
    nb = N_NODES // 5
    return pl.pallas_call(
        functools.partial(_update_body, beta),
        out_shape=(
            jax.ShapeDtypeStruct((N_NODES, C), jnp.float32),
            jax.ShapeDtypeStruct((N_NODES, 1), jnp.float32),
        ),
        grid=(5,),
        in_specs=[
            pl.BlockSpec((2, nb, C), lambda i: (0, i, 0)),
            pl.BlockSpec((2, nb, 1), lambda i: (0, i, 0)),
            pl.BlockSpec((nb, C), lambda i: (i, 0)),
            pl.BlockSpec((nb, C), lambda i: (i, 0)),
            pl.BlockSpec((nb, 2), lambda i: (i, 0)),
            pl.BlockSpec((nb, 1), lambda i: (i, 0)),
            pl.BlockSpec((nb, C), lambda i: (i, 0)),
            pl.BlockSpec((C, C), lambda i: (0, 0)),
        ],
        out_specs=(
            pl.BlockSpec((nb, C), lambda i: (i, 0)),
            pl.BlockSpec((nb, 1), lambda i: (i, 0)),
        ),
    )(aggpart, cpart, xp, xn, svec, dis, x0, wc)


def _head_body(h_ref, w_ref, b_ref, o_ref):
    logits = jnp.dot(h_ref[...], w_ref[...], preferred_element_type=jnp.float32)
    logits = logits + b_ref[...]
    m = jnp.max(logits, axis=-1, keepdims=True)
    z = logits - m
    lse = jnp.log(jnp.sum(jnp.exp(z), axis=-1, keepdims=True))
    o_ref[...] = z - lse


def _head(h, Wout, bout):
    nb = N_NODES // 5
    wp = jnp.zeros((C, 128), jnp.float32).at[:, :NUM_CLASSES].set(Wout)
    bp = jnp.full((1, 128), -1e30, jnp.float32).at[0, :NUM_CLASSES].set(bout)
    out = pl.pallas_call(
        _head_body,
        out_shape=jax.ShapeDtypeStruct((N_NODES, 128), jnp.float32),
        grid=(5,),
        in_specs=[
            pl.BlockSpec((nb, C), lambda i: (i, 0)),
            pl.BlockSpec((C, 128), lambda i: (0, 0)),
            pl.BlockSpec((1, 128), lambda i: (0, 0)),
        ],
        out_specs=pl.BlockSpec((nb, 128), lambda i: (i, 0)),
    )(h, wp, bp)
    return out[:, :NUM_CLASSES]


def kernel(x, edge_index, W0, b0, Wl, attl, attr, Wp, Wn, Wc, Wout, bout):
    pad = EPAD - N_EDGES
    srcp = jnp.concatenate([edge_index[0].astype(jnp.int32),
                            jnp.zeros((pad,), jnp.int32)])
    dstp = jnp.concatenate([edge_index[1].astype(jnp.int32),
                            jnp.zeros((pad,), jnp.int32)])
    onesp = jnp.concatenate([jnp.ones((N_EDGES,), jnp.float32),
                             jnp.zeros((pad,), jnp.float32)])
    zc = jnp.zeros((N_NODES,), jnp.float32)
    zagg = jnp.zeros((N_NODES, C), jnp.float32)

    degpart = _deg_kernel(dstp, onesp, zc)
    h, dis = _prolog(x, W0, b0, degpart.reshape(2, N_NODES, 1))
    ewp = _ew_kernel(srcp, dstp, onesp, dis.reshape(N_NODES))

    x0 = h
    sigmas = []
    for l in range(NUM_LAYERS):
        beta = math.log(THETA / (l + 1) + 1.0)
        attlr = jnp.stack([attl[l], attr[l]])
        xp, xn, svec = _proj(h, Wl[l], attlr, Wp[l], Wn[l])
        s_edges, cpart, aggpart = _edge_kernel(
            srcp, dstp, ewp,
            svec[:, 0], svec[:, 1],
            xp, zc, zagg)
        h, s_self = _update(beta, aggpart, cpart.reshape(2, N_NODES)
                            .reshape(2, N_NODES, 1),
                            xp, xn, svec, dis, x0, Wc[l])
        sigmas.append(jnp.concatenate([s_edges[:N_EDGES],
                                       s_self.reshape(N_NODES)]))
    return _head(h, Wout, bout), tuple(sigmas)
